# baseline (speedup 1.0000x reference)
import jax
import jax.numpy as jnp
from jax.experimental import pallas as pl


def _copy_body(x_ref, o_ref):
    o_ref[...] = x_ref[...]


def kernel(images):
    b, c, h, w = images.shape
    total = b * c * h * w
    lanes = 8192
    rows = total // lanes
    flat = images.reshape(rows, lanes)
    block_rows = 160
    out = pl.pallas_call(
        _copy_body,
        grid=(rows // block_rows,),
        in_specs=[pl.BlockSpec((block_rows, lanes), lambda i: (i, 0))],
        out_specs=pl.BlockSpec((block_rows, lanes), lambda i: (i, 0)),
        out_shape=jax.ShapeDtypeStruct((rows, lanes), jnp.float32),
    )(flat)
    return out.reshape(b, c, h, w)
